# Initial kernel scaffold; baseline (speedup 1.0000x reference)
import jax
import jax.numpy as jnp
from jax.experimental import pallas as pl


def kernel(x, edge_index, W1, b1, W2, b2):
    raise NotImplementedError("write your pallas kernel here")


# speedup vs baseline: 13.0719x; 13.0719x over previous
import functools

import jax
import jax.numpy as jnp
from jax import lax
from jax.experimental import pallas as pl
from jax.experimental.pallas import tpu as pltpu
from jax.experimental.pallas import tpu_sc as plsc

NS = 16
NC = 2
NW = NS * NC
CH = 128
BLK = 1024


def _zero_vmem(buf, rows, d):
    def row(i, _):
        def lane(j, _):
            buf[i, pl.ds(j * 16, 16)] = jnp.zeros((16,), jnp.float32)
            return 0
        return lax.fori_loop(0, d // 16, lane, 0)
    lax.fori_loop(0, rows, row, 0)


def _fill_ones(buf, rows, d):
    def row(i, _):
        def lane(j, _):
            buf[i, pl.ds(j * 16, 16)] = jnp.full((16,), 1.0, jnp.float32)
            return 0
        return lax.fori_loop(0, d // 16, lane, 0)
    lax.fori_loop(0, rows, row, 0)


@functools.lru_cache(maxsize=None)
def _make_sc_deg(NP, K):
    D = 16
    rows_per_tile = NP // NS
    mesh = plsc.VectorSubcoreMesh(core_axis_name="c", subcore_axis_name="s")

    @functools.partial(
        pl.kernel, mesh=mesh,
        out_type=jax.ShapeDtypeStruct((NC, NP, D), jnp.float32),
        scratch_types=[
            pltpu.VMEM((K, CH), jnp.int32),
            pltpu.VMEM((CH, D), jnp.float32),
            pltpu.VMEM_SHARED((NP, D), jnp.float32),
        ],
    )
    def k(dst_hbm, out_hbm, dst_v, ones_v, acc):
        c = lax.axis_index("c")
        s = lax.axis_index("s")
        wid = s * NC + c
        _zero_vmem(ones_v, CH, D)
        nz = rows_per_tile // CH
        def zcp(i, _):
            pltpu.sync_copy(ones_v, acc.at[pl.ds(s * rows_per_tile + i * CH, CH)])
            return 0
        lax.fori_loop(0, nz, zcp, 0)
        _fill_ones(ones_v, CH, D)
        plsc.subcore_barrier()
        pltpu.sync_copy(dst_hbm.at[wid], dst_v)
        def body(j, _):
            pltpu.sync_copy(ones_v, acc.at[dst_v.at[j]], add=True)
            return 0
        lax.fori_loop(0, K, body, 0)
        plsc.subcore_barrier()
        pltpu.sync_copy(acc.at[pl.ds(s * rows_per_tile, rows_per_tile)],
                        out_hbm.at[c].at[pl.ds(s * rows_per_tile, rows_per_tile)])

    return k


@functools.lru_cache(maxsize=None)
def _make_sc_scatter(NP, K, D):
    rows_per_tile = NP // NS
    mesh = plsc.VectorSubcoreMesh(core_axis_name="c", subcore_axis_name="s")

    @functools.partial(
        pl.kernel, mesh=mesh,
        out_type=jax.ShapeDtypeStruct((NC, NP, D), jnp.float32),
        scratch_types=[
            pltpu.VMEM((K, CH), jnp.int32),
            pltpu.VMEM((K, CH), jnp.int32),
            pltpu.VMEM((CH, D), jnp.float32),
            pltpu.VMEM_SHARED((NP, D), jnp.float32),
            pltpu.SemaphoreType.DMA,
        ],
    )
    def k(g_hbm, src_hbm, dst_hbm, out_hbm, src_v, dst_v, rows_v, acc, sem):
        c = lax.axis_index("c")
        s = lax.axis_index("s")
        wid = s * NC + c
        _zero_vmem(rows_v, CH, D)
        nz = rows_per_tile // CH
        def zcp(i, _):
            pltpu.sync_copy(rows_v, acc.at[pl.ds(s * rows_per_tile + i * CH, CH)])
            return 0
        lax.fori_loop(0, nz, zcp, 0)
        plsc.subcore_barrier()
        pltpu.sync_copy(src_hbm.at[wid], src_v)
        pltpu.sync_copy(dst_hbm.at[wid], dst_v)
        def body(j, _):
            pltpu.async_copy(g_hbm.at[src_v.at[j]], rows_v, sem).wait()
            pltpu.sync_copy(rows_v, acc.at[dst_v.at[j]], add=True)
            return 0
        lax.fori_loop(0, K, body, 0)
        plsc.subcore_barrier()
        pltpu.sync_copy(acc.at[pl.ds(s * rows_per_tile, rows_per_tile)],
                        out_hbm.at[c].at[pl.ds(s * rows_per_tile, rows_per_tile)])

    return k


def _dinv_from(dp_ref):
    deg = dp_ref[0, :, :1] + dp_ref[1, :, :1] + 1.0
    return lax.rsqrt(deg)


@functools.lru_cache(maxsize=None)
def _make_tc_a(NP, IN, HID):
    def body(x_ref, w_ref, dp_ref, o_ref):
        dinv = _dinv_from(dp_ref)
        o_ref[...] = jnp.dot(x_ref[...], w_ref[...],
                             preferred_element_type=jnp.float32) * dinv

    return pl.pallas_call(
        body,
        grid=(NP // BLK,),
        in_specs=[
            pl.BlockSpec((BLK, IN), lambda i: (i, 0)),
            pl.BlockSpec((IN, HID), lambda i: (0, 0)),
            pl.BlockSpec((NC, BLK, 16), lambda i: (0, i, 0)),
        ],
        out_specs=pl.BlockSpec((BLK, HID), lambda i: (i, 0)),
        out_shape=jax.ShapeDtypeStruct((NP, HID), jnp.float32),
    )


@functools.lru_cache(maxsize=None)
def _make_tc_b(NP, HID, LAT):
    def body(p_ref, g1_ref, dp_ref, w_ref, b_ref, o_ref):
        dinv = _dinv_from(dp_ref)
        ssum = p_ref[0] + p_ref[1] + g1_ref[...]
        h = jnp.maximum(ssum * dinv + b_ref[...], 0.0)
        o_ref[...] = jnp.dot(h, w_ref[...],
                             preferred_element_type=jnp.float32) * dinv

    return pl.pallas_call(
        body,
        grid=(NP // BLK,),
        in_specs=[
            pl.BlockSpec((NC, BLK, HID), lambda i: (0, i, 0)),
            pl.BlockSpec((BLK, HID), lambda i: (i, 0)),
            pl.BlockSpec((NC, BLK, 16), lambda i: (0, i, 0)),
            pl.BlockSpec((HID, LAT), lambda i: (0, 0)),
            pl.BlockSpec((1, HID), lambda i: (0, 0)),
        ],
        out_specs=pl.BlockSpec((BLK, LAT), lambda i: (i, 0)),
        out_shape=jax.ShapeDtypeStruct((NP, LAT), jnp.float32),
    )


@functools.lru_cache(maxsize=None)
def _make_tc_c(NP, LAT):
    def body(p_ref, g2_ref, dp_ref, b_ref, o_ref):
        dinv = _dinv_from(dp_ref)
        o_ref[...] = (p_ref[0] + p_ref[1] + g2_ref[...]) * dinv + b_ref[...]

    return pl.pallas_call(
        body,
        grid=(NP // BLK,),
        in_specs=[
            pl.BlockSpec((NC, BLK, LAT), lambda i: (0, i, 0)),
            pl.BlockSpec((BLK, LAT), lambda i: (i, 0)),
            pl.BlockSpec((NC, BLK, 16), lambda i: (0, i, 0)),
            pl.BlockSpec((1, LAT), lambda i: (0, 0)),
        ],
        out_specs=pl.BlockSpec((BLK, LAT), lambda i: (i, 0)),
        out_shape=jax.ShapeDtypeStruct((NP, LAT), jnp.float32),
    )


def kernel(x, edge_index, W1, b1, W2, b2):
    N, IN = x.shape
    HID = W1.shape[1]
    LAT = W2.shape[1]
    E = edge_index.shape[1]

    NP = ((N + 1 + BLK - 1) // BLK) * BLK
    K = (E + NW * CH - 1) // (NW * CH)
    EP = NW * K * CH

    LATP = max(128, ((LAT + 127) // 128) * 128)
    W2p = jnp.pad(W2, ((0, 0), (0, LATP - LAT)))
    b2p = jnp.pad(b2, ((0, LATP - LAT),))

    src = edge_index[0].astype(jnp.int32)
    dst = edge_index[1].astype(jnp.int32)
    pad = EP - E
    src_p = jnp.concatenate([src, jnp.zeros((pad,), jnp.int32)]).reshape(NW, K, CH)
    dst_p = jnp.concatenate([dst, jnp.full((pad,), N, jnp.int32)]).reshape(NW, K, CH)
    x_p = jnp.pad(x, ((0, NP - N), (0, 0)))

    degp = _make_sc_deg(NP, K)(dst_p)
    g1 = _make_tc_a(NP, IN, HID)(x_p, W1, degp)
    p1 = _make_sc_scatter(NP, K, HID)(g1, src_p, dst_p)
    g2 = _make_tc_b(NP, HID, LATP)(p1, g1, degp, W2p, b1.reshape(1, HID))
    p2 = _make_sc_scatter(NP, K, LATP)(g2, src_p, dst_p)
    z = _make_tc_c(NP, LATP)(p2, g2, degp, b2p.reshape(1, LATP))
    return z[:N, :LAT]
